# baseline (speedup 1.0000x reference)
import functools

import jax
import jax.numpy as jnp
from jax import lax
from jax.experimental import pallas as pl
from jax.experimental.pallas import tpu as pltpu
from jax.experimental.pallas import tpu_sc as plsc

VOCAB = 30522
D = 768
NUM_CLASS = 6
B = 1024
S = 128

NC, NS, L = 2, 16, 16
NW = NC * NS
BPW = B // NW
CH = 32
NV = D // L
PAD_C = 128


NB = 3
TOK = BPW * S
NCHMAX = TOK // CH


def _sc_pool(ids, mask, table):
    mesh = plsc.VectorSubcoreMesh(core_axis_name="c", subcore_axis_name="s")

    @functools.partial(
        pl.kernel,
        out_type=jax.ShapeDtypeStruct((B, D), jnp.float32),
        mesh=mesh,
        scratch_types=[
            pltpu.VMEM((BPW, S), jnp.int32),
            pltpu.VMEM((BPW, S), jnp.int32),
            pltpu.VMEM((NCHMAX, CH), jnp.int32),
            pltpu.VMEM((NCHMAX + L,), jnp.int32),
            pltpu.VMEM((BPW, D), jnp.float32),
            pltpu.VMEM((D,), jnp.float32),
            [pltpu.VMEM((CH, D), jnp.float32) for _ in range(NB)],
            [pltpu.SemaphoreType.DMA for _ in range(NB)],
        ],
        compiler_params=pltpu.CompilerParams(needs_layout_passes=False),
    )
    def k(ids_hbm, mask_hbm, table_hbm, out_hbm, ids_v, mask_v, comp_v, rowc_v,
          acc_v, row0_v, bufs, gsems):
        wid = lax.axis_index("s") * NC + lax.axis_index("c")
        base = wid * BPW
        zero_i = jnp.zeros((L,), jnp.int32)
        zero_f = jnp.zeros((L,), jnp.float32)
        iota = lax.iota(jnp.int32, L)

        pltpu.sync_copy(ids_hbm.at[pl.ds(base, BPW)], ids_v)
        pltpu.sync_copy(mask_hbm.at[pl.ds(base, BPW)], mask_v)
        pltpu.sync_copy(table_hbm.at[0], row0_v)

        def zero_body(i, _):
            for kk in range(NV):
                acc_v[i, pl.ds(kk * L, L)] = zero_f
            return 0

        lax.fori_loop(0, BPW, zero_body, 0)

        def comp_body(b, off):
            cur = off
            for j in range(S // L):
                iv = ids_v[b, pl.ds(j * L, L)]
                mv = mask_v[b, pl.ds(j * L, L)]
                cs = plsc.cumsum(mv)
                pos = cur + cs - mv
                plsc.store_scatter(comp_v, [pos >> 5, pos & 31], iv, mask=mv != 0)
                cur = cur + cs[L - 1]
            n_b = cur - off
            padc = (CH - lax.rem(n_b, CH)) % CH
            for q in range(CH // L):
                pp = cur + q * L + iota
                plsc.store_scatter(comp_v, [pp >> 5, pp & 31], zero_i,
                                   mask=iota + q * L < padc)
            pf = -padc.astype(jnp.float32)
            for kk in range(NV):
                plsc.addupdate(acc_v.at[b, pl.ds(kk * L, L)],
                               row0_v[pl.ds(kk * L, L)] * pf)
            c0 = off // CH
            ncb = (n_b + padc) // CH
            plsc.store_scatter(rowc_v, [c0 + iota], zero_i + b, mask=iota < ncb)
            return off + n_b + padc

        ntot = lax.fori_loop(0, BPW, comp_body, jnp.int32(0))
        nch = ntot // CH

        def issue(ci, b):
            pltpu.async_copy(table_hbm.at[comp_v.at[ci]], bufs[b], gsems[b])

        def wait_gather(ci, b):
            pltpu.make_async_copy(table_hbm.at[comp_v.at[ci]], bufs[b],
                                  gsems[b]).wait()

        for b in range(NB - 1):
            @pl.when(b < nch)
            def _(b=b):
                issue(b, b)

        ngrp = (nch + NB - 1) // NB

        def group_body(g, _):
            for b in range(NB):
                ci = g * NB + b

                @pl.when(ci < nch)
                def _(ci=ci, b=b):
                    wait_gather(ci, b)
                    cn = ci + NB - 1

                    @pl.when(cn < nch)
                    def _(cn=cn, b=b):
                        issue(cn, (b + NB - 1) % NB)

                    row = rowc_v[pl.ds(ci, L)][0]
                    HG = NV // 2
                    for h in range(2):
                        accs = tuple(acc_v[row, pl.ds((h * HG + u) * L, L)]
                                     for u in range(HG))

                        def tok_body(tt, accs, h=h, b=b):
                            return tuple(
                                accs[u] + bufs[b][tt, pl.ds((h * HG + u) * L, L)]
                                for u in range(HG))

                        accs = lax.fori_loop(0, CH, tok_body, accs)
                        for u in range(HG):
                            acc_v[row, pl.ds((h * HG + u) * L, L)] = accs[u]

            return 0

        lax.fori_loop(0, ngrp, group_body, 0)
        pltpu.sync_copy(acc_v, out_hbm.at[pl.ds(base, BPW)])

    return k(ids, mask, table)


def _tc_mlp(sums, mask, W1, b1, W2p, b2p):
    BB = 256

    def body(sum_ref, mask_ref, w1_ref, b1_ref, w2_ref, b2_ref, out_ref):
        cnt = jnp.sum(mask_ref[...].astype(jnp.float32), axis=1, keepdims=True)
        bow = sum_ref[...] / cnt
        h = jnp.dot(bow, w1_ref[...], preferred_element_type=jnp.float32) + b1_ref[...]
        h = jnp.maximum(h, 0.0)
        out_ref[...] = jnp.dot(h, w2_ref[...], preferred_element_type=jnp.float32) + b2_ref[...]

    return pl.pallas_call(
        body,
        grid=(B // BB,),
        in_specs=[
            pl.BlockSpec((BB, D), lambda i: (i, 0)),
            pl.BlockSpec((BB, S), lambda i: (i, 0)),
            pl.BlockSpec((D, D), lambda i: (0, 0)),
            pl.BlockSpec((1, D), lambda i: (0, 0)),
            pl.BlockSpec((D, PAD_C), lambda i: (0, 0)),
            pl.BlockSpec((1, PAD_C), lambda i: (0, 0)),
        ],
        out_specs=pl.BlockSpec((BB, PAD_C), lambda i: (i, 0)),
        out_shape=jax.ShapeDtypeStruct((B, PAD_C), jnp.float32),
    )(sums, mask, W1, b1.reshape(1, D), W2p, b2p.reshape(1, PAD_C))


def kernel(input_ids, attention_mask, emb_table, W1, b1, W2, b2):
    ids = input_ids.astype(jnp.int32)
    mask = attention_mask.astype(jnp.int32)
    sums = _sc_pool(ids, mask, emb_table)
    W2p = jnp.pad(W2, ((0, 0), (0, PAD_C - NUM_CLASS)))
    b2p = jnp.pad(b2, (0, PAD_C - NUM_CLASS))
    out = _tc_mlp(sums, mask, W1, b1, W2p, b2p)
    scores = out[:, :NUM_CLASS]
    return (scores, scores)


# speedup vs baseline: 2.7766x; 2.7766x over previous
import functools

import jax
import jax.numpy as jnp
from jax import lax
from jax.experimental import pallas as pl
from jax.experimental.pallas import tpu as pltpu
from jax.experimental.pallas import tpu_sc as plsc

VOCAB = 30522
D = 768
NUM_CLASS = 6
B = 1024
S = 128

NC, NS, L = 2, 16, 16
NW = NC * NS
BPW = B // NW
CH = 8
NV = D // L
PAD_C = 128


NB = 6
TOK = BPW * S
NCHMAX = TOK // CH


def _sc_pool(ids, mask, table):
    mesh = plsc.VectorSubcoreMesh(core_axis_name="c", subcore_axis_name="s")

    @functools.partial(
        pl.kernel,
        out_type=jax.ShapeDtypeStruct((B, D), jnp.float32),
        mesh=mesh,
        scratch_types=[
            pltpu.VMEM((BPW, S), jnp.int32),
            pltpu.VMEM((BPW, S), jnp.int32),
            pltpu.VMEM((TOK,), jnp.int32),
            pltpu.VMEM((NCHMAX + L,), jnp.int32),
            pltpu.VMEM((BPW, D), jnp.float32),
            pltpu.VMEM((D,), jnp.float32),
            [pltpu.VMEM((CH, D), jnp.float32) for _ in range(NB)],
            [pltpu.SemaphoreType.DMA for _ in range(NB)],
        ],
        compiler_params=pltpu.CompilerParams(needs_layout_passes=False),
    )
    def k(ids_hbm, mask_hbm, table_hbm, out_hbm, ids_v, mask_v, comp_v, rowc_v,
          acc_v, row0_v, bufs, gsems):
        wid = lax.axis_index("s") * NC + lax.axis_index("c")
        base = wid * BPW
        zero_i = jnp.zeros((L,), jnp.int32)
        zero_f = jnp.zeros((L,), jnp.float32)
        iota = lax.iota(jnp.int32, L)

        pltpu.sync_copy(ids_hbm.at[pl.ds(base, BPW)], ids_v)
        pltpu.sync_copy(mask_hbm.at[pl.ds(base, BPW)], mask_v)
        pltpu.sync_copy(table_hbm.at[0], row0_v)

        def zero_body(i, _):
            for kk in range(NV):
                acc_v[i, pl.ds(kk * L, L)] = zero_f
            return 0

        lax.fori_loop(0, BPW, zero_body, 0)

        def comp_body(b, off):
            cur = off
            for j in range(S // L):
                iv = ids_v[b, pl.ds(j * L, L)]
                mv = mask_v[b, pl.ds(j * L, L)]
                cs = plsc.cumsum(mv)
                pos = cur + cs - mv
                plsc.store_scatter(comp_v, [pos], iv, mask=mv != 0)
                cur = cur + cs[L - 1]
            n_b = cur - off
            padc = (CH - lax.rem(n_b, CH)) % CH
            pp = cur + iota
            plsc.store_scatter(comp_v, [pp], zero_i, mask=iota < padc)
            pf = -padc.astype(jnp.float32)
            for kk in range(NV):
                plsc.addupdate(acc_v.at[b, pl.ds(kk * L, L)],
                               row0_v[pl.ds(kk * L, L)] * pf)
            c0 = off // CH
            ncb = (n_b + padc) // CH
            plsc.store_scatter(rowc_v, [c0 + iota], zero_i + b, mask=iota < ncb)
            return off + n_b + padc

        ntot = lax.fori_loop(0, BPW, comp_body, jnp.int32(0))
        nch = ntot // CH

        def issue(ci, b):
            pltpu.async_copy(table_hbm.at[comp_v.at[pl.ds(ci * CH, CH)]],
                             bufs[b], gsems[b])

        def wait_gather(ci, b):
            pltpu.make_async_copy(table_hbm.at[comp_v.at[pl.ds(ci * CH, CH)]],
                                  bufs[b], gsems[b]).wait()

        for b in range(NB - 1):
            @pl.when(b < nch)
            def _(b=b):
                issue(b, b)

        ngrp = (nch + NB - 1) // NB

        def group_body(g, _):
            for b in range(NB):
                ci = g * NB + b

                @pl.when(ci < nch)
                def _(ci=ci, b=b):
                    wait_gather(ci, b)
                    cn = ci + NB - 1

                    @pl.when(cn < nch)
                    def _(cn=cn, b=b):
                        issue(cn, (b + NB - 1) % NB)

                    row = rowc_v[pl.ds(ci, L)][0]
                    HG = NV // 2
                    for h in range(2):
                        accs = tuple(acc_v[row, pl.ds((h * HG + u) * L, L)]
                                     for u in range(HG))

                        def tok_body(tt, accs, h=h, b=b):
                            return tuple(
                                accs[u] + bufs[b][tt, pl.ds((h * HG + u) * L, L)]
                                for u in range(HG))

                        accs = lax.fori_loop(0, CH, tok_body, accs)
                        for u in range(HG):
                            acc_v[row, pl.ds((h * HG + u) * L, L)] = accs[u]

            return 0

        lax.fori_loop(0, ngrp, group_body, 0)
        pltpu.sync_copy(acc_v, out_hbm.at[pl.ds(base, BPW)])

    return k(ids, mask, table)


def _tc_mlp(sums, mask, W1, b1, W2p, b2p):
    BB = 256

    def body(sum_ref, mask_ref, w1_ref, b1_ref, w2_ref, b2_ref, out_ref):
        cnt = jnp.sum(mask_ref[...].astype(jnp.float32), axis=1, keepdims=True)
        bow = sum_ref[...] / cnt
        h = jnp.dot(bow, w1_ref[...], preferred_element_type=jnp.float32) + b1_ref[...]
        h = jnp.maximum(h, 0.0)
        out_ref[...] = jnp.dot(h, w2_ref[...], preferred_element_type=jnp.float32) + b2_ref[...]

    return pl.pallas_call(
        body,
        grid=(B // BB,),
        in_specs=[
            pl.BlockSpec((BB, D), lambda i: (i, 0)),
            pl.BlockSpec((BB, S), lambda i: (i, 0)),
            pl.BlockSpec((D, D), lambda i: (0, 0)),
            pl.BlockSpec((1, D), lambda i: (0, 0)),
            pl.BlockSpec((D, PAD_C), lambda i: (0, 0)),
            pl.BlockSpec((1, PAD_C), lambda i: (0, 0)),
        ],
        out_specs=pl.BlockSpec((BB, PAD_C), lambda i: (i, 0)),
        out_shape=jax.ShapeDtypeStruct((B, PAD_C), jnp.float32),
    )(sums, mask, W1, b1.reshape(1, D), W2p, b2p.reshape(1, PAD_C))


def kernel(input_ids, attention_mask, emb_table, W1, b1, W2, b2):
    ids = input_ids.astype(jnp.int32)
    mask = attention_mask.astype(jnp.int32)
    sums = _sc_pool(ids, mask, emb_table)
    W2p = jnp.pad(W2, ((0, 0), (0, PAD_C - NUM_CLASS)))
    b2p = jnp.pad(b2, (0, PAD_C - NUM_CLASS))
    out = _tc_mlp(sums, mask, W1, b1, W2p, b2p)
    scores = out[:, :NUM_CLASS]
    return (scores, scores)
